# baseline (speedup 1.0000x reference)
import functools

import jax
import jax.numpy as jnp
from jax import lax
from jax.experimental import pallas as pl
from jax.experimental.pallas import tpu as pltpu
from jax.experimental.pallas import tpu_sc as plsc

K = 3
B, C, H, W = 4, 64, 128, 128
NCELL = B * H * W
NTAB = 3 * NCELL
ZROW = NTAB
NW = 32
CPT = NCELL // NW
CH = 16
NCHUNK = CPT // CH
NIDX = 3 * 16 * CH
NG = NIDX // 128
OUT_ELEMS = B * (K * H) * (K * W) * C


def _bcast_lane(v, i):
    idx = jnp.full((CH, 1), i, jnp.int32)
    return lax.gather(
        v, idx,
        lax.GatherDimensionNumbers(offset_dims=(), collapsed_slice_dims=(0,),
                                   start_index_map=(0,)),
        slice_sizes=(1,),
        mode=lax.GatherScatterMode.PROMISE_IN_BOUNDS)


def _floor(x):
    t = x.astype(jnp.int32)
    return jnp.where(t.astype(jnp.float32) > x, t - 1, t)


mesh = plsc.VectorSubcoreMesh(core_axis_name="c", subcore_axis_name="s")


@functools.partial(
    pl.kernel,
    mesh=mesh,
    compiler_params=pltpu.CompilerParams(use_tc_tiling_on_sc=False),
    out_type=jax.ShapeDtypeStruct((OUT_ELEMS,), jnp.float32),
    scratch_types=[
        pltpu.VMEM((3 * CPT,), jnp.float32),
        pltpu.VMEM((3 * CPT,), jnp.float32),
        pltpu.VMEM((3 * CPT,), jnp.float32),
        pltpu.VMEM((2 * NIDX,), jnp.int32),
        pltpu.VMEM((NIDX, C), jnp.float32),
        pltpu.VMEM((NIDX, C), jnp.float32),
        pltpu.VMEM((2 * 3 * 4 * CH,), jnp.float32),
        pltpu.VMEM((K * CH * K * C,), jnp.float32),
        pltpu.SemaphoreType.DMA,
        pltpu.SemaphoreType.DMA,
    ],
)
def _sc_extract(fxh, fyh, mh, table, out,
                fxv, fyv, mv, idxv, rows0, rows1, coefs, obuf, gsem0, gsem1):
    wid = lax.axis_index("s") * 2 + lax.axis_index("c")
    t0 = wid * CPT
    for s in range(3):
        pltpu.sync_copy(fxh.at[pl.ds(s * NCELL + t0, CPT)],
                        fxv.at[pl.ds(s * CPT, CPT)])
        pltpu.sync_copy(fyh.at[pl.ds(s * NCELL + t0, CPT)],
                        fyv.at[pl.ds(s * CPT, CPT)])
        pltpu.sync_copy(mh.at[pl.ds(s * NCELL + t0, CPT)],
                        mv.at[pl.ds(s * CPT, CPT)])

    lanes = lax.iota(jnp.int32, CH)

    def cell_of(ch):
        cell0 = t0 + ch * CH
        b = cell0 // (H * W)
        rem = cell0 - b * (H * W)
        hf = rem // W
        wf0 = rem - hf * W
        return b, hf, wf0

    def compute_idx(ch, pbuf):
        b, hf, wf0 = cell_of(ch)
        wfv = (wf0 + lanes).astype(jnp.float32)
        hfs = hf.astype(jnp.float32)
        ibase = pbuf * NIDX
        cbase = pbuf * 12 * CH
        for s in range(3):
            off = s * CPT + ch * CH
            fxc = fxv[pl.ds(off, CH)]
            fyc = fyv[pl.ds(off, CH)]
            mc = mv[pl.ds(off, CH)]
            xc = wfv + fxc
            yc = hfs + fyc
            x0 = _floor(xc)
            y0 = _floor(yc)
            fxf = xc - x0.astype(jnp.float32)
            fyf = yc - y0.astype(jnp.float32)
            my0 = mc * (1.0 - fyf)
            my1 = mc * fyf
            coefs[pl.ds(cbase + (s * 4 + 0) * CH, CH)] = my0 * (1.0 - fxf)
            coefs[pl.ds(cbase + (s * 4 + 1) * CH, CH)] = my0 * fxf
            coefs[pl.ds(cbase + (s * 4 + 2) * CH, CH)] = my1 * (1.0 - fxf)
            coefs[pl.ds(cbase + (s * 4 + 3) * CH, CH)] = my1 * fxf
            sbase = s * NCELL + b * (H * W)
            for gy in range(4):
                yi = y0 + (gy - 1)
                vy = (yi >= 0) & (yi < H)
                yterm = sbase + yi * W
                for gx in range(4):
                    xi = x0 + (gx - 1)
                    valid = vy & (xi >= 0) & (xi < W)
                    idx = jnp.where(valid, yterm + xi, ZROW)
                    slot = s * 16 + gy * 4 + gx
                    idxv[pl.ds(ibase + slot * CH, CH)] = idx

    def fire(pbuf, rows, gsem):
        ibase = pbuf * NIDX
        for g in range(NG):
            pltpu.async_copy(table.at[idxv.at[pl.ds(ibase + g * 128, 128)]],
                             rows.at[pl.ds(g * 128, 128)], gsem)

    def drain(rows, gsem):
        for g in range(NG):
            pltpu.make_async_copy(table.at[pl.ds(0, 128)],
                                  rows.at[pl.ds(g * 128, 128)], gsem).wait()

    def combine(ch, pbuf, rows):
        b, hf, wf0 = cell_of(ch)
        cbase = pbuf * 12 * CH

        def cell_body(i, carry2):
            cs = [[_bcast_lane(coefs[pl.ds(cbase + (s * 4 + ab) * CH, CH)], i)
                   for ab in range(4)] for s in range(3)]
            for ci in range(C // 16):
                acc = [[None] * K for _ in range(K)]
                for s in range(3):
                    for gy in range(4):
                        for gx in range(4):
                            slot = s * 16 + gy * 4 + gx
                            v = rows[slot * CH + i, pl.ds(ci * 16, 16)]
                            for ky in (gy - 1, gy):
                                if not 0 <= ky <= 2:
                                    continue
                                for kx in (gx - 1, gx):
                                    if not 0 <= kx <= 2:
                                        continue
                                    cab = cs[s][(gy - ky) * 2 + (gx - kx)]
                                    term = cab * v
                                    if acc[ky][kx] is None:
                                        acc[ky][kx] = term
                                    else:
                                        acc[ky][kx] = acc[ky][kx] + term
                for ky in range(K):
                    for kx in range(K):
                        obuf[pl.ds(ky * CH * K * C + (i * K + kx) * C
                                   + ci * 16, 16)] = acc[ky][kx]
            return carry2

        lax.fori_loop(0, CH, cell_body, 0)

        for ky in range(K):
            dst = ((b * K * H + hf * K + ky) * (K * W) + wf0 * K) * C
            pltpu.sync_copy(obuf.at[pl.ds(ky * CH * K * C, CH * K * C)],
                            out.at[pl.ds(dst, CH * K * C)])

    compute_idx(0, 0)
    fire(0, rows0, gsem0)

    def pair_body(ch2, carry):
        e = ch2 * 2
        compute_idx(e + 1, 1)
        fire(1, rows1, gsem1)
        drain(rows0, gsem0)
        combine(e, 0, rows0)
        compute_idx((e + 2) & (NCHUNK - 1), 0)
        fire(0, rows0, gsem0)
        drain(rows1, gsem1)
        combine(e + 1, 1, rows1)
        return carry

    lax.fori_loop(0, NCHUNK // 2, pair_body, 0)
    drain(rows0, gsem0)


def kernel(source_a, source_b, source_c,
           flow_field_a, flow_field_b, flow_field_c,
           masks_a, masks_b, masks_c):
    def rows_of(s):
        return jnp.transpose(s, (0, 2, 3, 1)).reshape(NCELL, C)

    table = jnp.concatenate(
        [rows_of(source_a), rows_of(source_b), rows_of(source_c),
         jnp.zeros((8, C), jnp.float32)], axis=0)
    fx = jnp.stack([flow_field_a[:, 0], flow_field_b[:, 0],
                    flow_field_c[:, 0]]).reshape(3 * NCELL)
    fy = jnp.stack([flow_field_a[:, 1], flow_field_b[:, 1],
                    flow_field_c[:, 1]]).reshape(3 * NCELL)
    mm = jnp.stack([masks_a[:, 0], masks_b[:, 0],
                    masks_c[:, 0]]).reshape(3 * NCELL)
    out_flat = _sc_extract(fx, fy, mm, table)
    return out_flat.reshape(B, K * H, K * W, C).transpose(0, 3, 1, 2)


# speedup vs baseline: 1.3542x; 1.3542x over previous
import functools

import jax
import jax.numpy as jnp
from jax import lax
from jax.experimental import pallas as pl
from jax.experimental.pallas import tpu as pltpu
from jax.experimental.pallas import tpu_sc as plsc

K = 3
B, C, H, W = 4, 64, 128, 128
NCELL = B * H * W
WX = W + 8
QW = 4 * C
ZROW = 3 * B * H * WX
NW = 32
CPT = NCELL // NW
CH = 16
NCHUNK = CPT // CH
NIDX = 3 * 4 * CH
OUT_ELEMS = B * (K * H) * (K * W) * C


def _bcast_lane(v, i):
    idx = jnp.full((CH, 1), i, jnp.int32)
    return lax.gather(
        v, idx,
        lax.GatherDimensionNumbers(offset_dims=(), collapsed_slice_dims=(0,),
                                   start_index_map=(0,)),
        slice_sizes=(1,),
        mode=lax.GatherScatterMode.PROMISE_IN_BOUNDS)


def _floor(x):
    t = x.astype(jnp.int32)
    return jnp.where(t.astype(jnp.float32) > x, t - 1, t)


mesh = plsc.VectorSubcoreMesh(core_axis_name="c", subcore_axis_name="s")


@functools.partial(
    pl.kernel,
    mesh=mesh,
    compiler_params=pltpu.CompilerParams(use_tc_tiling_on_sc=False),
    out_type=jax.ShapeDtypeStruct((OUT_ELEMS,), jnp.float32),
    scratch_types=[
        pltpu.VMEM((3 * CPT,), jnp.float32),
        pltpu.VMEM((3 * CPT,), jnp.float32),
        pltpu.VMEM((3 * CPT,), jnp.float32),
        pltpu.VMEM((2 * NIDX,), jnp.int32),
        pltpu.VMEM((NIDX, QW), jnp.float32),
        pltpu.VMEM((NIDX, QW), jnp.float32),
        pltpu.VMEM((2 * 3 * 4 * CH,), jnp.float32),
        pltpu.VMEM((K * CH * K * C,), jnp.float32),
        pltpu.SemaphoreType.DMA,
        pltpu.SemaphoreType.DMA,
    ],
)
def _sc_extract(fxh, fyh, mh, table, out,
                fxv, fyv, mv, idxv, rows0, rows1, coefs, obuf, gsem0, gsem1):
    wid = lax.axis_index("s") * 2 + lax.axis_index("c")
    t0 = wid * CPT
    for s in range(3):
        pltpu.sync_copy(fxh.at[pl.ds(s * NCELL + t0, CPT)],
                        fxv.at[pl.ds(s * CPT, CPT)])
        pltpu.sync_copy(fyh.at[pl.ds(s * NCELL + t0, CPT)],
                        fyv.at[pl.ds(s * CPT, CPT)])
        pltpu.sync_copy(mh.at[pl.ds(s * NCELL + t0, CPT)],
                        mv.at[pl.ds(s * CPT, CPT)])

    lanes = lax.iota(jnp.int32, CH)

    def cell_of(ch):
        cell0 = t0 + ch * CH
        b = cell0 // (H * W)
        rem = cell0 - b * (H * W)
        hf = rem // W
        wf0 = rem - hf * W
        return b, hf, wf0

    def compute_idx(ch, pbuf):
        b, hf, wf0 = cell_of(ch)
        wfv = (wf0 + lanes).astype(jnp.float32)
        hfs = hf.astype(jnp.float32)
        ibase = pbuf * NIDX
        cbase = pbuf * 12 * CH
        for s in range(3):
            off = s * CPT + ch * CH
            fxc = fxv[pl.ds(off, CH)]
            fyc = fyv[pl.ds(off, CH)]
            mc = mv[pl.ds(off, CH)]
            xc = wfv + fxc
            yc = hfs + fyc
            x0 = _floor(xc)
            y0 = _floor(yc)
            fxf = xc - x0.astype(jnp.float32)
            fyf = yc - y0.astype(jnp.float32)
            my0 = mc * (1.0 - fyf)
            my1 = mc * fyf
            coefs[pl.ds(cbase + (s * 4 + 0) * CH, CH)] = my0 * (1.0 - fxf)
            coefs[pl.ds(cbase + (s * 4 + 1) * CH, CH)] = my0 * fxf
            coefs[pl.ds(cbase + (s * 4 + 2) * CH, CH)] = my1 * (1.0 - fxf)
            coefs[pl.ds(cbase + (s * 4 + 3) * CH, CH)] = my1 * fxf
            xl = x0 - 1
            vx = (xl >= -3) & (xl <= W)
            sbase = (s * B + b) * (H * WX)
            for gy in range(4):
                yi = y0 + (gy - 1)
                valid = vx & (yi >= 0) & (yi < H)
                idx = jnp.where(valid, sbase + yi * WX + (xl + 3), ZROW)
                slot = s * 4 + gy
                idxv[pl.ds(ibase + slot * CH, CH)] = idx

    def fire(pbuf, rows, gsem):
        ibase = pbuf * NIDX
        pltpu.async_copy(table.at[idxv.at[pl.ds(ibase, 128)]],
                         rows.at[pl.ds(0, 128)], gsem)
        pltpu.async_copy(table.at[idxv.at[pl.ds(ibase + 128, 64)]],
                         rows.at[pl.ds(128, 64)], gsem)

    def drain(rows, gsem):
        pltpu.make_async_copy(table.at[pl.ds(0, 128)],
                              rows.at[pl.ds(0, 128)], gsem).wait()
        pltpu.make_async_copy(table.at[pl.ds(0, 64)],
                              rows.at[pl.ds(128, 64)], gsem).wait()

    def combine(ch, pbuf, rows):
        b, hf, wf0 = cell_of(ch)
        cbase = pbuf * 12 * CH

        def cell_body(i, carry2):
            cs = [[_bcast_lane(coefs[pl.ds(cbase + (s * 4 + ab) * CH, CH)], i)
                   for ab in range(4)] for s in range(3)]
            for ci in range(C // 16):
                acc = [[None] * K for _ in range(K)]
                for s in range(3):
                    for gy in range(4):
                        slot = s * 4 + gy
                        for gx in range(4):
                            v = rows[slot * CH + i,
                                     pl.ds(gx * 64 + ci * 16, 16)]
                            for ky in (gy - 1, gy):
                                if not 0 <= ky <= 2:
                                    continue
                                for kx in (gx - 1, gx):
                                    if not 0 <= kx <= 2:
                                        continue
                                    cab = cs[s][(gy - ky) * 2 + (gx - kx)]
                                    term = cab * v
                                    if acc[ky][kx] is None:
                                        acc[ky][kx] = term
                                    else:
                                        acc[ky][kx] = acc[ky][kx] + term
                for ky in range(K):
                    for kx in range(K):
                        obuf[pl.ds(ky * CH * K * C + (i * K + kx) * C
                                   + ci * 16, 16)] = acc[ky][kx]
            return carry2

        lax.fori_loop(0, CH, cell_body, 0)

        for ky in range(K):
            dst = ((b * K * H + hf * K + ky) * (K * W) + wf0 * K) * C
            pltpu.sync_copy(obuf.at[pl.ds(ky * CH * K * C, CH * K * C)],
                            out.at[pl.ds(dst, CH * K * C)])

    compute_idx(0, 0)
    fire(0, rows0, gsem0)

    def pair_body(ch2, carry):
        e = ch2 * 2
        compute_idx(e + 1, 1)
        fire(1, rows1, gsem1)
        drain(rows0, gsem0)
        combine(e, 0, rows0)
        compute_idx((e + 2) & (NCHUNK - 1), 0)
        fire(0, rows0, gsem0)
        drain(rows1, gsem1)
        combine(e + 1, 1, rows1)
        return carry

    lax.fori_loop(0, NCHUNK // 2, pair_body, 0)
    drain(rows0, gsem0)


def kernel(source_a, source_b, source_c,
           flow_field_a, flow_field_b, flow_field_c,
           masks_a, masks_b, masks_c):
    def rows_of(s):
        st = jnp.transpose(s, (0, 2, 3, 1))
        pad = jnp.pad(st, ((0, 0), (0, 0), (3, 8), (0, 0)))
        win = jnp.stack([pad[:, :, j:j + WX] for j in range(4)], axis=3)
        return win.reshape(B * H * WX, QW)

    table = jnp.concatenate(
        [rows_of(source_a), rows_of(source_b), rows_of(source_c),
         jnp.zeros((8, QW), jnp.float32)], axis=0)
    fx = jnp.stack([flow_field_a[:, 0], flow_field_b[:, 0],
                    flow_field_c[:, 0]]).reshape(3 * NCELL)
    fy = jnp.stack([flow_field_a[:, 1], flow_field_b[:, 1],
                    flow_field_c[:, 1]]).reshape(3 * NCELL)
    mm = jnp.stack([masks_a[:, 0], masks_b[:, 0],
                    masks_c[:, 0]]).reshape(3 * NCELL)
    out_flat = _sc_extract(fx, fy, mm, table)
    return out_flat.reshape(B, K * H, K * W, C).transpose(0, 3, 1, 2)
